# baseline (speedup 1.0000x reference)
import functools

import jax
import jax.numpy as jnp
from jax import lax
from jax.experimental import pallas as pl
from jax.experimental.pallas import tpu as pltpu
from jax.experimental.pallas import tpu_sc as plsc

VOCAB = 1000000
EMBED_DIM = 64
BATCH = 4096
SEQ = 200

NC = 2
NS = 16
NW = NC * NS
B_BLK = BATCH // NW
N_CHUNKS = SEQ
N_PAIRS = N_CHUNKS // 2

V_BLK = 64
N_VBLK = VOCAB // V_BLK
VBLK_BASE = N_VBLK // NW
VBLK_REM = N_VBLK - VBLK_BASE * NW

_SC_PARAMS = pltpu.CompilerParams(
    use_tc_tiling_on_sc=False, needs_layout_passes=False
)


def _worker_id():
    return lax.axis_index("s") * NC + lax.axis_index("c")


def _make_rots():
    lane = lax.iota(jnp.int32, 16)
    rots = []
    r = lane
    for _ in range(16):
        rots.append(r)
        r = jnp.bitwise_and(r + 1, 15)
    return lane, rots


@functools.partial(
    pl.kernel,
    out_type=jax.ShapeDtypeStruct((VOCAB, EMBED_DIM), jnp.float32),
    mesh=plsc.VectorSubcoreMesh(core_axis_name="c", subcore_axis_name="s"),
    scratch_types=[
        pltpu.VMEM((EMBED_DIM, V_BLK), jnp.float32),
        pltpu.VMEM((EMBED_DIM, V_BLK), jnp.float32),
        pltpu.VMEM((V_BLK, EMBED_DIM), jnp.float32),
        pltpu.VMEM((V_BLK, EMBED_DIM), jnp.float32),
        pltpu.SemaphoreType.DMA,
        pltpu.SemaphoreType.DMA,
        pltpu.SemaphoreType.DMA,
        pltpu.SemaphoreType.DMA,
    ],
    compiler_params=_SC_PARAMS,
)
def _relayout_kernel(wt_hbm, table_hbm, in0, in1, t0, t1,
                     si0, si1, so0, so1):
    wid = _worker_id()
    nblk = VBLK_BASE + jnp.where(wid < VBLK_REM, 1, 0)
    start = wid * VBLK_BASE + jnp.minimum(wid, VBLK_REM)
    inb = (in0, in1)
    tbuf = (t0, t1)
    si = (si0, si1)
    so = (so0, so1)
    lane, rots = _make_rots()

    def fire_in(g, b):
        v0 = (start + g) * V_BLK
        pltpu.async_copy(wt_hbm.at[:, pl.ds(v0, V_BLK)], inb[b], si[b])

    def wait_in(b):
        pltpu.make_async_copy(wt_hbm.at[:, pl.ds(0, V_BLK)], inb[b],
                              si[b]).wait()

    def transpose(b):
        @plsc.parallel_loop(0, 16, unroll=2)
        def _(i):
            r0 = (i % 4) * 16
            c0 = (i // 4) * 16
            rowv = lane + r0
            for d in range(16):
                colv = rots[d] + c0
                v = plsc.load_gather(inb[b], [rowv, colv])
                plsc.store_scatter(tbuf[b], [colv, rowv], v)

    def fire_out(g, b):
        v0 = (start + g) * V_BLK
        pltpu.async_copy(tbuf[b], table_hbm.at[pl.ds(v0, V_BLK)], so[b])

    def wait_out(b):
        pltpu.make_async_copy(tbuf[b], table_hbm.at[pl.ds(0, V_BLK)],
                              so[b]).wait()

    fire_in(0, 0)
    fire_in(1, 1)

    def body(i, carry):
        for b in (0, 1):
            g = 2 * i + b
            @pl.when(g < nblk)
            def _():
                wait_in(b)
                @pl.when(i > 0)
                def _():
                    wait_out(b)
                transpose(b)
                fire_out(g, b)
                @pl.when(g + 2 < nblk)
                def _():
                    fire_in(g + 2, b)
        return carry

    lax.fori_loop(0, (VBLK_BASE + 2) // 2, body, 0)
    wait_out(0)
    wait_out(1)


@functools.partial(
    pl.kernel,
    out_type=jax.ShapeDtypeStruct((SEQ, EMBED_DIM // 8, NW, 8, B_BLK),
                                  jnp.float32),
    mesh=plsc.VectorSubcoreMesh(core_axis_name="c", subcore_axis_name="s"),
    scratch_types=[
        pltpu.VMEM((N_CHUNKS, B_BLK), jnp.int32),
        pltpu.VMEM((B_BLK, EMBED_DIM), jnp.float32),
        pltpu.VMEM((B_BLK, EMBED_DIM), jnp.float32),
        pltpu.VMEM((EMBED_DIM, B_BLK), jnp.float32),
        pltpu.VMEM((EMBED_DIM, B_BLK), jnp.float32),
        pltpu.SemaphoreType.DMA,
        pltpu.SemaphoreType.DMA,
        pltpu.SemaphoreType.DMA,
        pltpu.SemaphoreType.DMA,
    ],
    compiler_params=_SC_PARAMS,
)
def _gather_kernel(idx_hbm, table_hbm, out_hbm, idx_v, rows0, rows1,
                   t0, t1, sg0, sg1, ss0, ss1):
    wid = _worker_id()
    rows = (rows0, rows1)
    tbuf = (t0, t1)
    sg = (sg0, sg1)
    ss = (ss0, ss1)
    lane, rots = _make_rots()

    pltpu.sync_copy(idx_hbm.at[pl.ds(wid * N_CHUNKS, N_CHUNKS)], idx_v)

    def fire_gather(g, b):
        pltpu.async_copy(table_hbm.at[idx_v.at[g]], rows[b], sg[b])

    def wait_gather(b):
        pltpu.make_async_copy(table_hbm.at[idx_v.at[0]], rows[b],
                              sg[b]).wait()

    def transpose(b):
        @plsc.parallel_loop(0, 32, unroll=2)
        def _(i):
            r0 = (i % 8) * 16
            c0 = (i // 8) * 16
            rowv = lane + r0
            for d in range(16):
                colv = rots[d] + c0
                v = plsc.load_gather(rows[b], [rowv, colv])
                plsc.store_scatter(tbuf[b], [colv, rowv], v)

    def fire_store(g, b):
        for eh in range(EMBED_DIM // 8):
            pltpu.async_copy(tbuf[b].at[pl.ds(8 * eh, 8)],
                             out_hbm.at[g, eh, wid], ss[b])

    def wait_store(b):
        for eh in range(EMBED_DIM // 8):
            pltpu.make_async_copy(tbuf[b].at[pl.ds(8 * eh, 8)],
                                  out_hbm.at[0, eh, wid], ss[b]).wait()

    fire_gather(0, 0)
    fire_gather(1, 1)

    def body(i, carry):
        for b in (0, 1):
            g = 2 * i + b
            wait_gather(b)
            @pl.when(i > 0)
            def _():
                wait_store(b)
            transpose(b)
            fire_store(g, b)
            @pl.when(i < N_PAIRS - 1)
            def _():
                fire_gather(g + 2, b)
        return carry

    lax.fori_loop(0, N_PAIRS, body, 0)
    wait_store(0)
    wait_store(1)


def kernel(input_, weight):
    idx_r = (input_.reshape(NW, B_BLK, SEQ)
             .transpose(0, 2, 1)
             .reshape(NW * SEQ, B_BLK))
    table = _relayout_kernel(weight.T)
    out5 = _gather_kernel(idx_r, table)
    return out5.transpose(2, 4, 0, 1, 3).reshape(BATCH, SEQ, EMBED_DIM)


# speedup vs baseline: 7.8468x; 7.8468x over previous
import functools

import jax
import jax.numpy as jnp
from jax import lax
from jax.experimental import pallas as pl
from jax.experimental.pallas import tpu as pltpu
from jax.experimental.pallas import tpu_sc as plsc

VOCAB = 1000000
EMBED_DIM = 64
PAD_DIM = 128
BATCH = 4096
SEQ = 200

NC = 2
NS = 16
NW = NC * NS
B_BLK = BATCH // NW
N_CHUNKS = SEQ
N_PAIRS = N_CHUNKS // 2
EH = EMBED_DIM // 8


@functools.partial(
    pl.kernel,
    out_type=jax.ShapeDtypeStruct((SEQ, EH, NW, 8, B_BLK), jnp.float32),
    mesh=plsc.VectorSubcoreMesh(core_axis_name="c", subcore_axis_name="s"),
    scratch_types=[
        pltpu.VMEM((N_CHUNKS, B_BLK), jnp.int32),
        pltpu.VMEM((B_BLK, EMBED_DIM), jnp.float32),
        pltpu.VMEM((B_BLK, EMBED_DIM), jnp.float32),
        pltpu.VMEM((EMBED_DIM, B_BLK), jnp.float32),
        pltpu.VMEM((EMBED_DIM, B_BLK), jnp.float32),
        pltpu.SemaphoreType.DMA,
        pltpu.SemaphoreType.DMA,
        pltpu.SemaphoreType.DMA,
        pltpu.SemaphoreType.DMA,
    ],
    compiler_params=pltpu.CompilerParams(
        use_tc_tiling_on_sc=False, needs_layout_passes=False
    ),
)
def _gather_kernel(idx_hbm, table_hbm, out_hbm, idx_v, rows0, rows1,
                   t0, t1, sg0, sg1, ss0, ss1):
    wid = lax.axis_index("s") * NC + lax.axis_index("c")
    rows = (rows0, rows1)
    tbuf = (t0, t1)
    sg = (sg0, sg1)
    ss = (ss0, ss1)

    pltpu.sync_copy(idx_hbm.at[pl.ds(wid * N_CHUNKS, N_CHUNKS)], idx_v)

    lane = lax.iota(jnp.int32, 16)
    rots = []
    _r = lane
    for _ in range(16):
        rots.append(_r)
        _r = jnp.bitwise_and(_r + 1, 15)

    def fire_gather(g, b):
        pltpu.async_copy(table_hbm.at[idx_v.at[g]], rows[b], sg[b])

    def wait_gather(b):
        pltpu.make_async_copy(table_hbm.at[idx_v.at[0]], rows[b],
                              sg[b]).wait()

    def transpose(b):
        @plsc.parallel_loop(0, 32, unroll=4)
        def _(i):
            r0 = (i % 8) * 16
            c0 = (i // 8) * 16
            rowv = lane + r0
            for d in range(16):
                colv = rots[d] + c0
                v = plsc.load_gather(rows[b], [rowv, colv])
                plsc.store_scatter(tbuf[b], [colv, rowv], v)

    def fire_store(g, b):
        for eh in range(EH):
            pltpu.async_copy(tbuf[b].at[pl.ds(8 * eh, 8)],
                             out_hbm.at[g, eh, wid], ss[b])

    def wait_store(b):
        for eh in range(EH):
            pltpu.make_async_copy(tbuf[b].at[pl.ds(8 * eh, 8)],
                                  out_hbm.at[0, eh, wid], ss[b]).wait()

    fire_gather(0, 0)
    fire_gather(1, 1)

    def body(i, carry):
        for b in (0, 1):
            g = 2 * i + b
            wait_gather(b)
            @pl.when(i > 0)
            def _():
                wait_store(b)
            transpose(b)
            fire_store(g, b)
            @pl.when(i < N_PAIRS - 1)
            def _():
                fire_gather(g + 2, b)
        return carry

    lax.fori_loop(0, N_PAIRS, body, 0)
    wait_store(0)
    wait_store(1)


def kernel(input_, weight):
    idx_r = (input_.reshape(NW, B_BLK, SEQ)
             .transpose(0, 2, 1)
             .reshape(NW * SEQ, B_BLK)) * 2
    wpad = jnp.pad(weight, ((0, 0), (0, PAD_DIM - EMBED_DIM)))
    table2 = wpad.reshape(2 * VOCAB, EMBED_DIM)
    out5 = _gather_kernel(idx_r, table2)
    return out5.transpose(2, 4, 0, 1, 3).reshape(BATCH, SEQ, EMBED_DIM)


# speedup vs baseline: 8.2050x; 1.0457x over previous
import functools

import jax
import jax.numpy as jnp
from jax import lax
from jax.experimental import pallas as pl
from jax.experimental.pallas import tpu as pltpu
from jax.experimental.pallas import tpu_sc as plsc

VOCAB = 1000000
EMBED_DIM = 64
PAD_DIM = 128
BATCH = 4096
SEQ = 200

NC = 2
NS = 16
NW = NC * NS
B_BLK = BATCH // NW
N_CHUNKS = SEQ
N_PAIRS = N_CHUNKS // 2
EH = EMBED_DIM // 8


@functools.partial(
    pl.kernel,
    out_type=jax.ShapeDtypeStruct((SEQ, EH, NW, 8, B_BLK), jnp.float32),
    mesh=plsc.VectorSubcoreMesh(core_axis_name="c", subcore_axis_name="s"),
    scratch_types=[
        pltpu.VMEM((N_CHUNKS, B_BLK), jnp.int32),
        pltpu.VMEM((B_BLK, EMBED_DIM), jnp.float32),
        pltpu.VMEM((B_BLK, EMBED_DIM), jnp.float32),
        pltpu.VMEM((B_BLK, EMBED_DIM), jnp.float32),
        pltpu.VMEM((EMBED_DIM, B_BLK), jnp.float32),
        pltpu.VMEM((EMBED_DIM, B_BLK), jnp.float32),
        pltpu.VMEM((EMBED_DIM, B_BLK), jnp.float32),
        pltpu.SemaphoreType.DMA,
        pltpu.SemaphoreType.DMA,
        pltpu.SemaphoreType.DMA,
        pltpu.SemaphoreType.DMA,
        pltpu.SemaphoreType.DMA,
        pltpu.SemaphoreType.DMA,
    ],
    compiler_params=pltpu.CompilerParams(
        use_tc_tiling_on_sc=False, needs_layout_passes=False
    ),
)
def _gather_kernel(idx_hbm, table_hbm, out_hbm, idx_v, rows0, rows1, rows2,
                   t0, t1, t2, sg0, sg1, sg2, ss0, ss1, ss2):
    wid = lax.axis_index("s") * NC + lax.axis_index("c")
    rows = (rows0, rows1, rows2)
    tbuf = (t0, t1, t2)
    sg = (sg0, sg1, sg2)
    ss = (ss0, ss1, ss2)

    pltpu.sync_copy(idx_hbm.at[pl.ds(wid * N_CHUNKS, N_CHUNKS)], idx_v)

    lane = lax.iota(jnp.int32, 16)
    rots = []
    _r = lane
    for _ in range(16):
        rots.append(_r)
        _r = jnp.bitwise_and(_r + 1, 15)

    def fire_gather(g, b):
        pltpu.async_copy(table_hbm.at[idx_v.at[g]], rows[b], sg[b])

    def wait_gather(b):
        pltpu.make_async_copy(table_hbm.at[idx_v.at[0]], rows[b],
                              sg[b]).wait()

    def transpose(b):
        @plsc.parallel_loop(0, 32, unroll=4)
        def _(i):
            r0 = (i % 8) * 16
            c0 = (i // 8) * 16
            rowv = lane + r0
            for d in range(16):
                colv = rots[d] + c0
                v = plsc.load_gather(rows[b], [rowv, colv])
                plsc.store_scatter(tbuf[b], [colv, rowv], v)

    def fire_store(g, b):
        for eh in range(EH):
            pltpu.async_copy(tbuf[b].at[pl.ds(8 * eh, 8)],
                             out_hbm.at[g, eh, wid], ss[b])

    def wait_store(b):
        for eh in range(EH):
            pltpu.make_async_copy(tbuf[b].at[pl.ds(8 * eh, 8)],
                                  out_hbm.at[0, eh, wid], ss[b]).wait()

    fire_gather(0, 0)
    fire_gather(1, 1)
    fire_gather(2, 2)

    def body(i, carry):
        for b in (0, 1, 2):
            g = 3 * i + b
            @pl.when(g < N_CHUNKS)
            def _():
                wait_gather(b)
                @pl.when(i > 0)
                def _():
                    wait_store(b)
                transpose(b)
                fire_store(g, b)
                @pl.when(g + 3 < N_CHUNKS)
                def _():
                    fire_gather(g + 3, b)
        return carry

    lax.fori_loop(0, (N_CHUNKS + 2) // 3, body, 0)
    wait_store(0)
    wait_store(1)
    wait_store(2)


def kernel(input_, weight):
    idx_r = (input_.reshape(NW, B_BLK, SEQ)
             .transpose(0, 2, 1)
             .reshape(NW * SEQ, B_BLK)) * 2
    wpad = jnp.pad(weight, ((0, 0), (0, PAD_DIM - EMBED_DIM)))
    table2 = wpad.reshape(2 * VOCAB, EMBED_DIM)
    out5 = _gather_kernel(idx_r, table2)
    return out5.transpose(2, 4, 0, 1, 3).reshape(BATCH, SEQ, EMBED_DIM)
